# baseline (speedup 1.0000x reference)
import dataclasses
import functools

import jax
import jax.numpy as jnp
from jax import lax
from jax.experimental import pallas as pl
from jax.experimental.pallas import tpu as pltpu
from jax.experimental.pallas import tpu_sc as plsc

C = 10000
N = 320000
D = 128
HALF = C // 2
PAD = 5120
RPT = PAD // 16
K = 64
NCH = N // K
SUB = 128
NSUB = K // SUB
CH_PER_TILE = (NCH + 15) // 16
HROW = 48


def _sc_stats(x, labels):
    mesh = plsc.VectorSubcoreMesh(core_axis_name="c", subcore_axis_name="s")
    out_type = (
        jax.ShapeDtypeStruct((2, PAD, D), jnp.float32),
        jax.ShapeDtypeStruct((2, PAD, D), jnp.float32),
        jax.ShapeDtypeStruct((2, HROW, 128), jnp.float32),
    )

    cp = pltpu.CompilerParams()
    if "needs_layout_passes" in pltpu.CompilerParams.__dataclass_fields__:
        cp = dataclasses.replace(cp, needs_layout_passes=False)

    @functools.partial(
        pl.kernel,
        mesh=mesh,
        out_type=out_type,
        compiler_params=cp,
        scratch_types=[
            pltpu.VMEM((K, D), jnp.float32),
            pltpu.VMEM((K, D), jnp.float32),
            pltpu.VMEM((K, D), jnp.float32),
            pltpu.VMEM((K, D), jnp.float32),
            pltpu.VMEM((K,), jnp.int32),
            pltpu.VMEM((K,), jnp.int32),
            pltpu.VMEM((1, K), jnp.int32),
            pltpu.VMEM((1, K), jnp.int32),
            pltpu.VMEM((HROW, 128), jnp.float32),
            pltpu.VMEM((HROW,), jnp.int32),
            pltpu.SMEM((4,), jnp.int32),
            pltpu.SemaphoreType.DMA,
            pltpu.SemaphoreType.DMA,
            pltpu.SemaphoreType.DMA,
            pltpu.SemaphoreType.DMA,
            pltpu.SemaphoreType.DMA,
            pltpu.SemaphoreType.DMA,
            pltpu.VMEM_SHARED((PAD, D), jnp.float32),
            pltpu.VMEM_SHARED((PAD, D), jnp.float32),
            pltpu.VMEM_SHARED((HROW, 128), jnp.float32),
        ],
    )
    def k(x_hbm, lbl_hbm, sum_hbm, sq_hbm, cnt_hbm,
          xb0, xb1, sq0, sq1, lb0, lb1, ix0, ix1, hist, iota, st,
          sem_l0, sem_l1, sem_s0, sem_s1, sem_x0, sem_x1,
          acc_s, acc_q, acc_c):
        core = lax.axis_index("c")
        sub = lax.axis_index("s")
        xb = (xb0, xb1)
        sq = (sq0, sq1)
        lb = (lb0, lb1)
        ix = (ix0, ix1)
        sem_l = (sem_l0, sem_l1)
        sem_s = (sem_s0, sem_s1)
        sem_x = (sem_x0, sem_x1)

        @pl.loop(0, K)
        def _(r):
            z = jnp.zeros((16,), jnp.float32)
            for j in range(D // 16):
                xb0[r, pl.ds(j * 16, 16)] = z

        @pl.loop(0, HROW)
        def _(r):
            z = jnp.zeros((16,), jnp.float32)
            for j in range(D // 16):
                hist[r, pl.ds(j * 16, 16)] = z

        @pl.loop(0, HROW, step=16)
        def _(r):
            iota[pl.ds(r, 16)] = r + lax.iota(jnp.int32, 16)

        st[0] = 0
        st[1] = 0

        r0 = sub * RPT
        for h in range(RPT // K):
            pltpu.sync_copy(xb0, acc_s.at[pl.ds(r0 + h * K, K)])
            pltpu.sync_copy(xb0, acc_q.at[pl.ds(r0 + h * K, K)])

        @pl.when(sub == 0)
        def _():
            pltpu.sync_copy(hist, acc_c)

        plsc.subcore_barrier()

        def _active(lbp):
            first = lbp[pl.ds(0, 16)][0]
            last = lbp[pl.ds(K - 16, 16)][15]
            act = jnp.where(core == 0, first < HALF, last >= HALF)
            return jnp.where(act, 1, 0)

        pltpu.async_copy(lbl_hbm.at[pl.ds(sub * K, K)], lb0, sem_l0)
        pltpu.async_copy(lbl_hbm.at[pl.ds((sub + 16) * K, K)], lb1, sem_l1)
        pltpu.make_async_copy(lbl_hbm.at[pl.ds(0, K)], lb0, sem_l0).wait()
        st[2] = _active(lb0)

        @pl.when(st[2] == 1)
        def _():
            pltpu.async_copy(x_hbm.at[pl.ds(sub * K, K)], xb0, sem_x0)

        @pl.loop(0, (CH_PER_TILE + 1) // 2)
        def _(jj):
            for p in range(2):
                kk = jj * 2 + p
                ci = sub + kk * 16
                ci_n = ci + 16
                xbp, sqp, lbp, ixp = xb[p], sq[p], lb[p], ix[p]

                @pl.when(ci_n < NCH)
                def _():
                    pltpu.make_async_copy(lbl_hbm.at[pl.ds(0, K)], lb[1 - p],
                                          sem_l[1 - p]).wait()
                    actn = _active(lb[1 - p])
                    st[3 - p] = actn

                    @pl.when(actn == 1)
                    def _():
                        @pl.when(st[1 - p] == 1)
                        def _():
                            pltpu.make_async_copy(x_hbm.at[pl.ds(0, K)],
                                                  xb[1 - p],
                                                  sem_s[1 - p]).wait()
                            pltpu.make_async_copy(x_hbm.at[pl.ds(0, K)],
                                                  sq[1 - p],
                                                  sem_s[1 - p]).wait()
                            st[1 - p] = 0

                        pltpu.async_copy(x_hbm.at[pl.ds(ci_n * K, K)],
                                         xb[1 - p], sem_x[1 - p])

                @pl.when((ci < NCH) & (st[2 + p] == 1))
                def _():
                    pltpu.make_async_copy(x_hbm.at[pl.ds(0, K)], xbp,
                                          sem_x[p]).wait()
                    for cc in range(0, K, 16):
                        v = lbp[pl.ds(cc, 16)]
                        w = jnp.where(core == 0,
                                      jnp.minimum(v, HALF),
                                      jnp.maximum(v - (HALF - 1), 0))
                        ixp[0, pl.ds(cc, 16)] = w
                        plsc.addupdate_scatter(
                            hist, [w >> 7, w & 127],
                            jnp.ones((16,), jnp.float32))

                    pltpu.async_copy(xbp, acc_s.at[ixp.at[0]], sem_s[p],
                                     add=True)

                    @plsc.parallel_loop(0, K, unroll=8)
                    def _(r):
                        for j in range(D // 16):
                            v = xbp[r, pl.ds(j * 16, 16)]
                            sqp[r, pl.ds(j * 16, 16)] = v * v

                    pltpu.async_copy(sqp, acc_q.at[ixp.at[0]], sem_s[p],
                                     add=True)
                    st[p] = 1

                @pl.when(ci + 32 < NCH)
                def _():
                    pltpu.async_copy(lbl_hbm.at[pl.ds((ci + 32) * K, K)],
                                     lbp, sem_l[p])

        for p in range(2):
            @pl.when(st[p] == 1)
            def _(p=p):
                pltpu.make_async_copy(x_hbm.at[pl.ds(0, K)], xb[p],
                                      sem_s[p]).wait()
                pltpu.make_async_copy(x_hbm.at[pl.ds(0, K)], sq[p],
                                      sem_s[p]).wait()

        plsc.subcore_barrier()
        pltpu.sync_copy(hist, acc_c.at[iota], add=True)
        plsc.subcore_barrier()

        for h in range(RPT // K):
            rr = r0 + h * K
            pltpu.sync_copy(acc_s.at[pl.ds(rr, K)], xb0)
            pltpu.sync_copy(xb0, sum_hbm.at[core, pl.ds(rr, K)])
            pltpu.sync_copy(acc_q.at[pl.ds(rr, K)], xb1)
            pltpu.sync_copy(xb1, sq_hbm.at[core, pl.ds(rr, K)])

        @pl.when(sub == 0)
        def _():
            pltpu.sync_copy(acc_c.at[pl.ds(0, HROW)], hist)
            pltpu.sync_copy(hist, cnt_hbm.at[core])

    return k(x, labels)


_RB = 2560


def _mm_body(x_ref, w_ref, b_ref, o_ref):
    o_ref[...] = jnp.dot(x_ref[...], w_ref[...],
                         preferred_element_type=jnp.float32) + b_ref[...]


def _matmul(x, w, b2):
    return pl.pallas_call(
        _mm_body,
        grid=(N // _RB,),
        in_specs=[pl.BlockSpec((_RB, D), lambda i: (i, 0)),
                  pl.BlockSpec((D, D), lambda i: (0, 0)),
                  pl.BlockSpec((1, D), lambda i: (0, 0))],
        out_specs=pl.BlockSpec((_RB, D), lambda i: (i, 0)),
        out_shape=jax.ShapeDtypeStruct((N, D), jnp.float32),
    )(x, w, b2)


_FB = 2000


def _fin_body(s_ref, q_ref, c1_ref, rm_ref, rv_ref, cc_ref,
              om_ref, ov_ref, oc_ref):
    cnt = c1_ref[...]
    safe_new = jnp.maximum(cnt, 1.0)
    mean_new = s_ref[...] / safe_new
    var_new = q_ref[...] / safe_new - mean_new * mean_new
    pos = cnt > 0
    mean_new = jnp.where(pos, mean_new, 0.0)
    var_new = jnp.where(pos, var_new, 0.0)
    n_a = cc_ref[...]
    rm = rm_ref[...]
    nn = n_a + cnt
    safe = jnp.maximum(nn, 1.0)
    mean = (n_a * rm + cnt * mean_new) / safe
    var = (n_a * (rv_ref[...] + (rm - mean) ** 2)
           + cnt * (var_new + (mean_new - mean) ** 2)) / safe
    ok = nn > 0
    om_ref[...] = jnp.where(ok, mean, 0.0)
    ov_ref[...] = jnp.where(ok, var, 0.0)
    oc_ref[...] = nn


def _finalize(sums, sqs, c1, running_mean, running_var, class_count):
    return pl.pallas_call(
        _fin_body,
        grid=(C // _FB,),
        in_specs=[pl.BlockSpec((_FB, D), lambda i: (i, 0)),
                  pl.BlockSpec((_FB, D), lambda i: (i, 0)),
                  pl.BlockSpec((_FB, 1), lambda i: (i, 0)),
                  pl.BlockSpec((_FB, D), lambda i: (i, 0)),
                  pl.BlockSpec((_FB, D), lambda i: (i, 0)),
                  pl.BlockSpec((_FB, 1), lambda i: (i, 0))],
        out_specs=[pl.BlockSpec((_FB, D), lambda i: (i, 0)),
                   pl.BlockSpec((_FB, D), lambda i: (i, 0)),
                   pl.BlockSpec((_FB, 1), lambda i: (i, 0))],
        out_shape=[jax.ShapeDtypeStruct((C, D), jnp.float32),
                   jax.ShapeDtypeStruct((C, D), jnp.float32),
                   jax.ShapeDtypeStruct((C, 1), jnp.float32)],
    )(sums, sqs, c1, running_mean, running_var, class_count)


def kernel(x, labels, W, b, running_mean, running_var, class_count):
    labels = labels.astype(jnp.int32)
    sums2, sqs2, cnt2 = _sc_stats(x, labels)
    out = _matmul(x, W, b.reshape(1, D))
    sums = jnp.concatenate([sums2[0, :HALF], sums2[1, 1:HALF + 1]], axis=0)
    sqs = jnp.concatenate([sqs2[0, :HALF], sqs2[1, 1:HALF + 1]], axis=0)
    cflat0 = cnt2[0].reshape(HROW * 128)[:HALF]
    cflat1 = cnt2[1].reshape(HROW * 128)[1:HALF + 1]
    c1 = jnp.concatenate([cflat0, cflat1], axis=0)[:, None]
    rm, rv, cc = _finalize(sums, sqs, c1, running_mean, running_var,
                           class_count)
    return out, rm, rv, cc


# speedup vs baseline: 1.0159x; 1.0159x over previous
import dataclasses
import functools

import jax
import jax.numpy as jnp
from jax import lax
from jax.experimental import pallas as pl
from jax.experimental.pallas import tpu as pltpu
from jax.experimental.pallas import tpu_sc as plsc

C = 10000
N = 320000
D = 128
HALF = C // 2
PAD = 5120
RPT = PAD // 16
K = 80
NCH = N // K
SUB = 128
NSUB = K // SUB
CH_PER_TILE = (NCH + 15) // 16
HROW = 48


def _sc_stats(x, labels):
    mesh = plsc.VectorSubcoreMesh(core_axis_name="c", subcore_axis_name="s")
    out_type = (
        jax.ShapeDtypeStruct((2, PAD, D), jnp.float32),
        jax.ShapeDtypeStruct((2, PAD, D), jnp.float32),
        jax.ShapeDtypeStruct((2, HROW, 128), jnp.float32),
    )

    cp = pltpu.CompilerParams()
    if "needs_layout_passes" in pltpu.CompilerParams.__dataclass_fields__:
        cp = dataclasses.replace(cp, needs_layout_passes=False)

    @functools.partial(
        pl.kernel,
        mesh=mesh,
        out_type=out_type,
        compiler_params=cp,
        scratch_types=[
            pltpu.VMEM((K, D), jnp.float32),
            pltpu.VMEM((K, D), jnp.float32),
            pltpu.VMEM((K, D), jnp.float32),
            pltpu.VMEM((K, D), jnp.float32),
            pltpu.VMEM((K,), jnp.int32),
            pltpu.VMEM((K,), jnp.int32),
            pltpu.VMEM((1, K), jnp.int32),
            pltpu.VMEM((1, K), jnp.int32),
            pltpu.VMEM((HROW, 128), jnp.float32),
            pltpu.VMEM((HROW,), jnp.int32),
            pltpu.SMEM((4,), jnp.int32),
            pltpu.SemaphoreType.DMA,
            pltpu.SemaphoreType.DMA,
            pltpu.SemaphoreType.DMA,
            pltpu.SemaphoreType.DMA,
            pltpu.SemaphoreType.DMA,
            pltpu.SemaphoreType.DMA,
            pltpu.VMEM_SHARED((PAD, D), jnp.float32),
            pltpu.VMEM_SHARED((PAD, D), jnp.float32),
            pltpu.VMEM_SHARED((HROW, 128), jnp.float32),
        ],
    )
    def k(x_hbm, lbl_hbm, sum_hbm, sq_hbm, cnt_hbm,
          xb0, xb1, sq0, sq1, lb0, lb1, ix0, ix1, hist, iota, st,
          sem_l0, sem_l1, sem_s0, sem_s1, sem_x0, sem_x1,
          acc_s, acc_q, acc_c):
        core = lax.axis_index("c")
        sub = lax.axis_index("s")
        xb = (xb0, xb1)
        sq = (sq0, sq1)
        lb = (lb0, lb1)
        ix = (ix0, ix1)
        sem_l = (sem_l0, sem_l1)
        sem_s = (sem_s0, sem_s1)
        sem_x = (sem_x0, sem_x1)

        @pl.loop(0, K)
        def _(r):
            z = jnp.zeros((16,), jnp.float32)
            for j in range(D // 16):
                xb0[r, pl.ds(j * 16, 16)] = z

        @pl.loop(0, HROW)
        def _(r):
            z = jnp.zeros((16,), jnp.float32)
            for j in range(D // 16):
                hist[r, pl.ds(j * 16, 16)] = z

        @pl.loop(0, HROW, step=16)
        def _(r):
            iota[pl.ds(r, 16)] = r + lax.iota(jnp.int32, 16)

        st[0] = 0
        st[1] = 0

        r0 = sub * RPT
        for h in range(RPT // K):
            pltpu.sync_copy(xb0, acc_s.at[pl.ds(r0 + h * K, K)])
            pltpu.sync_copy(xb0, acc_q.at[pl.ds(r0 + h * K, K)])

        @pl.when(sub == 0)
        def _():
            pltpu.sync_copy(hist, acc_c)

        plsc.subcore_barrier()

        def _active(lbp):
            first = lbp[pl.ds(0, 16)][0]
            last = lbp[pl.ds(K - 16, 16)][15]
            act = jnp.where(core == 0, first < HALF, last >= HALF)
            return jnp.where(act, 1, 0)

        pltpu.async_copy(lbl_hbm.at[pl.ds(sub * K, K)], lb0, sem_l0)
        pltpu.async_copy(lbl_hbm.at[pl.ds((sub + 16) * K, K)], lb1, sem_l1)
        pltpu.make_async_copy(lbl_hbm.at[pl.ds(0, K)], lb0, sem_l0).wait()
        st[2] = _active(lb0)

        @pl.when(st[2] == 1)
        def _():
            pltpu.async_copy(x_hbm.at[pl.ds(sub * K, K)], xb0, sem_x0)

        @pl.loop(0, (CH_PER_TILE + 1) // 2)
        def _(jj):
            for p in range(2):
                kk = jj * 2 + p
                ci = sub + kk * 16
                ci_n = ci + 16
                xbp, sqp, lbp, ixp = xb[p], sq[p], lb[p], ix[p]

                @pl.when(ci_n < NCH)
                def _():
                    pltpu.make_async_copy(lbl_hbm.at[pl.ds(0, K)], lb[1 - p],
                                          sem_l[1 - p]).wait()
                    actn = _active(lb[1 - p])
                    st[3 - p] = actn

                    @pl.when(actn == 1)
                    def _():
                        @pl.when(st[1 - p] == 1)
                        def _():
                            pltpu.make_async_copy(x_hbm.at[pl.ds(0, K)],
                                                  xb[1 - p],
                                                  sem_s[1 - p]).wait()
                            pltpu.make_async_copy(x_hbm.at[pl.ds(0, K)],
                                                  sq[1 - p],
                                                  sem_s[1 - p]).wait()
                            st[1 - p] = 0

                        pltpu.async_copy(x_hbm.at[pl.ds(ci_n * K, K)],
                                         xb[1 - p], sem_x[1 - p])

                @pl.when((ci < NCH) & (st[2 + p] == 1))
                def _():
                    pltpu.make_async_copy(x_hbm.at[pl.ds(0, K)], xbp,
                                          sem_x[p]).wait()
                    for cc in range(0, K, 16):
                        v = lbp[pl.ds(cc, 16)]
                        w = jnp.where(core == 0,
                                      jnp.minimum(v, HALF),
                                      jnp.maximum(v - (HALF - 1), 0))
                        ixp[0, pl.ds(cc, 16)] = w
                        plsc.addupdate_scatter(
                            hist, [w >> 7, w & 127],
                            jnp.ones((16,), jnp.float32))

                    pltpu.async_copy(xbp, acc_s.at[ixp.at[0]], sem_s[p],
                                     add=True)

                    @plsc.parallel_loop(0, K, unroll=8)
                    def _(r):
                        for j in range(D // 16):
                            v = xbp[r, pl.ds(j * 16, 16)]
                            sqp[r, pl.ds(j * 16, 16)] = v * v

                    pltpu.async_copy(sqp, acc_q.at[ixp.at[0]], sem_s[p],
                                     add=True)
                    st[p] = 1

                @pl.when(ci + 32 < NCH)
                def _():
                    pltpu.async_copy(lbl_hbm.at[pl.ds((ci + 32) * K, K)],
                                     lbp, sem_l[p])

        for p in range(2):
            @pl.when(st[p] == 1)
            def _(p=p):
                pltpu.make_async_copy(x_hbm.at[pl.ds(0, K)], xb[p],
                                      sem_s[p]).wait()
                pltpu.make_async_copy(x_hbm.at[pl.ds(0, K)], sq[p],
                                      sem_s[p]).wait()

        plsc.subcore_barrier()
        pltpu.sync_copy(hist, acc_c.at[iota], add=True)
        plsc.subcore_barrier()

        for h in range(RPT // K):
            rr = r0 + h * K
            pltpu.sync_copy(acc_s.at[pl.ds(rr, K)], xb0)
            pltpu.sync_copy(xb0, sum_hbm.at[core, pl.ds(rr, K)])
            pltpu.sync_copy(acc_q.at[pl.ds(rr, K)], xb1)
            pltpu.sync_copy(xb1, sq_hbm.at[core, pl.ds(rr, K)])

        @pl.when(sub == 0)
        def _():
            pltpu.sync_copy(acc_c.at[pl.ds(0, HROW)], hist)
            pltpu.sync_copy(hist, cnt_hbm.at[core])

    return k(x, labels)


_RB = 2560


def _mm_body(x_ref, w_ref, b_ref, o_ref):
    o_ref[...] = jnp.dot(x_ref[...], w_ref[...],
                         preferred_element_type=jnp.float32) + b_ref[...]


def _matmul(x, w, b2):
    return pl.pallas_call(
        _mm_body,
        grid=(N // _RB,),
        in_specs=[pl.BlockSpec((_RB, D), lambda i: (i, 0)),
                  pl.BlockSpec((D, D), lambda i: (0, 0)),
                  pl.BlockSpec((1, D), lambda i: (0, 0))],
        out_specs=pl.BlockSpec((_RB, D), lambda i: (i, 0)),
        out_shape=jax.ShapeDtypeStruct((N, D), jnp.float32),
    )(x, w, b2)


_FB = 2000


def _fin_body(s_ref, q_ref, c1_ref, rm_ref, rv_ref, cc_ref,
              om_ref, ov_ref, oc_ref):
    cnt = c1_ref[...]
    safe_new = jnp.maximum(cnt, 1.0)
    mean_new = s_ref[...] / safe_new
    var_new = q_ref[...] / safe_new - mean_new * mean_new
    pos = cnt > 0
    mean_new = jnp.where(pos, mean_new, 0.0)
    var_new = jnp.where(pos, var_new, 0.0)
    n_a = cc_ref[...]
    rm = rm_ref[...]
    nn = n_a + cnt
    safe = jnp.maximum(nn, 1.0)
    mean = (n_a * rm + cnt * mean_new) / safe
    var = (n_a * (rv_ref[...] + (rm - mean) ** 2)
           + cnt * (var_new + (mean_new - mean) ** 2)) / safe
    ok = nn > 0
    om_ref[...] = jnp.where(ok, mean, 0.0)
    ov_ref[...] = jnp.where(ok, var, 0.0)
    oc_ref[...] = nn


def _finalize(sums, sqs, c1, running_mean, running_var, class_count):
    return pl.pallas_call(
        _fin_body,
        grid=(C // _FB,),
        in_specs=[pl.BlockSpec((_FB, D), lambda i: (i, 0)),
                  pl.BlockSpec((_FB, D), lambda i: (i, 0)),
                  pl.BlockSpec((_FB, 1), lambda i: (i, 0)),
                  pl.BlockSpec((_FB, D), lambda i: (i, 0)),
                  pl.BlockSpec((_FB, D), lambda i: (i, 0)),
                  pl.BlockSpec((_FB, 1), lambda i: (i, 0))],
        out_specs=[pl.BlockSpec((_FB, D), lambda i: (i, 0)),
                   pl.BlockSpec((_FB, D), lambda i: (i, 0)),
                   pl.BlockSpec((_FB, 1), lambda i: (i, 0))],
        out_shape=[jax.ShapeDtypeStruct((C, D), jnp.float32),
                   jax.ShapeDtypeStruct((C, D), jnp.float32),
                   jax.ShapeDtypeStruct((C, 1), jnp.float32)],
    )(sums, sqs, c1, running_mean, running_var, class_count)


def kernel(x, labels, W, b, running_mean, running_var, class_count):
    labels = labels.astype(jnp.int32)
    sums2, sqs2, cnt2 = _sc_stats(x, labels)
    out = _matmul(x, W, b.reshape(1, D))
    sums = jnp.concatenate([sums2[0, :HALF], sums2[1, 1:HALF + 1]], axis=0)
    sqs = jnp.concatenate([sqs2[0, :HALF], sqs2[1, 1:HALF + 1]], axis=0)
    cflat0 = cnt2[0].reshape(HROW * 128)[:HALF]
    cflat1 = cnt2[1].reshape(HROW * 128)[1:HALF + 1]
    c1 = jnp.concatenate([cflat0, cflat1], axis=0)[:, None]
    rm, rv, cc = _finalize(sums, sqs, c1, running_mean, running_var,
                           class_count)
    return out, rm, rv, cc


# speedup vs baseline: 1.0162x; 1.0002x over previous
import dataclasses
import functools

import jax
import jax.numpy as jnp
from jax import lax
from jax.experimental import pallas as pl
from jax.experimental.pallas import tpu as pltpu
from jax.experimental.pallas import tpu_sc as plsc

C = 10000
N = 320000
D = 128
HALF = C // 2
PAD = 5120
RPT = PAD // 16
K = 80
NCH = N // K
SUB = 128
NSUB = K // SUB
CH_PER_TILE = (NCH + 15) // 16
HROW = 48


def _sc_stats(x, labels):
    mesh = plsc.VectorSubcoreMesh(core_axis_name="c", subcore_axis_name="s")
    out_type = (
        jax.ShapeDtypeStruct((2, PAD, D), jnp.float32),
        jax.ShapeDtypeStruct((2, PAD, D), jnp.float32),
        jax.ShapeDtypeStruct((2, HROW, 128), jnp.float32),
    )

    cp = pltpu.CompilerParams()
    if "needs_layout_passes" in pltpu.CompilerParams.__dataclass_fields__:
        cp = dataclasses.replace(cp, needs_layout_passes=False)

    @functools.partial(
        pl.kernel,
        mesh=mesh,
        out_type=out_type,
        compiler_params=cp,
        scratch_types=[
            pltpu.VMEM((K, D), jnp.float32),
            pltpu.VMEM((K, D), jnp.float32),
            pltpu.VMEM((K, D), jnp.float32),
            pltpu.VMEM((K, D), jnp.float32),
            pltpu.VMEM((K,), jnp.int32),
            pltpu.VMEM((K,), jnp.int32),
            pltpu.VMEM((1, K), jnp.int32),
            pltpu.VMEM((1, K), jnp.int32),
            pltpu.VMEM((HROW, 128), jnp.float32),
            pltpu.VMEM((HROW,), jnp.int32),
            pltpu.SMEM((4,), jnp.int32),
            pltpu.SemaphoreType.DMA,
            pltpu.SemaphoreType.DMA,
            pltpu.SemaphoreType.DMA,
            pltpu.SemaphoreType.DMA,
            pltpu.SemaphoreType.DMA,
            pltpu.SemaphoreType.DMA,
            pltpu.VMEM_SHARED((PAD, D), jnp.float32),
            pltpu.VMEM_SHARED((PAD, D), jnp.float32),
            pltpu.VMEM_SHARED((HROW, 128), jnp.float32),
        ],
    )
    def k(x_hbm, lbl_hbm, sum_hbm, sq_hbm, cnt_hbm,
          xb0, xb1, sq0, sq1, lb0, lb1, ix0, ix1, hist, iota, st,
          sem_l0, sem_l1, sem_s0, sem_s1, sem_x0, sem_x1,
          acc_s, acc_q, acc_c):
        core = lax.axis_index("c")
        sub = lax.axis_index("s")
        xb = (xb0, xb1)
        sq = (sq0, sq1)
        lb = (lb0, lb1)
        ix = (ix0, ix1)
        sem_l = (sem_l0, sem_l1)
        sem_s = (sem_s0, sem_s1)
        sem_x = (sem_x0, sem_x1)

        @pl.loop(0, K)
        def _(r):
            z = jnp.zeros((16,), jnp.float32)
            for j in range(D // 16):
                xb0[r, pl.ds(j * 16, 16)] = z

        @pl.loop(0, HROW)
        def _(r):
            z = jnp.zeros((16,), jnp.float32)
            for j in range(D // 16):
                hist[r, pl.ds(j * 16, 16)] = z

        @pl.loop(0, HROW, step=16)
        def _(r):
            iota[pl.ds(r, 16)] = r + lax.iota(jnp.int32, 16)

        st[0] = 0
        st[1] = 0

        r0 = sub * RPT
        for h in range(RPT // K):
            pltpu.sync_copy(xb0, acc_s.at[pl.ds(r0 + h * K, K)])
            pltpu.sync_copy(xb0, acc_q.at[pl.ds(r0 + h * K, K)])

        @pl.when(sub == 0)
        def _():
            pltpu.sync_copy(hist, acc_c)

        plsc.subcore_barrier()

        def _active(lbp):
            first = lbp[pl.ds(0, 16)][0]
            last = lbp[pl.ds(K - 16, 16)][15]
            act = jnp.where(core == 0, first < HALF, last >= HALF)
            return jnp.where(act, 1, 0)

        pltpu.async_copy(lbl_hbm.at[pl.ds(sub * K, K)], lb0, sem_l0)
        pltpu.async_copy(lbl_hbm.at[pl.ds((sub + 16) * K, K)], lb1, sem_l1)
        pltpu.make_async_copy(lbl_hbm.at[pl.ds(0, K)], lb0, sem_l0).wait()
        st[2] = _active(lb0)

        @pl.when(st[2] == 1)
        def _():
            pltpu.async_copy(x_hbm.at[pl.ds(sub * K, K)], xb0, sem_x0)

        @pl.loop(0, (CH_PER_TILE + 1) // 2)
        def _(jj):
            for p in range(2):
                kk = jj * 2 + p
                ci = sub + kk * 16
                ci_n = ci + 16
                xbp, sqp, lbp, ixp = xb[p], sq[p], lb[p], ix[p]

                @pl.when(ci_n < NCH)
                def _():
                    pltpu.make_async_copy(lbl_hbm.at[pl.ds(0, K)], lb[1 - p],
                                          sem_l[1 - p]).wait()
                    actn = _active(lb[1 - p])
                    st[3 - p] = actn

                    @pl.when(actn == 1)
                    def _():
                        @pl.when(st[1 - p] == 1)
                        def _():
                            pltpu.make_async_copy(x_hbm.at[pl.ds(0, K)],
                                                  xb[1 - p],
                                                  sem_s[1 - p]).wait()
                            pltpu.make_async_copy(x_hbm.at[pl.ds(0, K)],
                                                  sq[1 - p],
                                                  sem_s[1 - p]).wait()
                            st[1 - p] = 0

                        pltpu.async_copy(x_hbm.at[pl.ds(ci_n * K, K)],
                                         xb[1 - p], sem_x[1 - p])

                @pl.when((ci < NCH) & (st[2 + p] == 1))
                def _():
                    pltpu.make_async_copy(x_hbm.at[pl.ds(0, K)], xbp,
                                          sem_x[p]).wait()
                    for cc in range(0, K, 16):
                        v = lbp[pl.ds(cc, 16)]
                        w = jnp.where(core == 0,
                                      jnp.minimum(v, HALF),
                                      jnp.maximum(v - (HALF - 1), 0))
                        ixp[0, pl.ds(cc, 16)] = w
                        plsc.addupdate_scatter(
                            hist, [w >> 7, w & 127],
                            jnp.ones((16,), jnp.float32))

                    pltpu.async_copy(xbp, acc_s.at[ixp.at[0]], sem_s[p],
                                     add=True)

                    @plsc.parallel_loop(0, K, unroll=8)
                    def _(r):
                        for j in range(D // 16):
                            v = xbp[r, pl.ds(j * 16, 16)]
                            sqp[r, pl.ds(j * 16, 16)] = v * v

                    pltpu.async_copy(sqp, acc_q.at[ixp.at[0]], sem_s[p],
                                     add=True)
                    st[p] = 1

                @pl.when(ci + 32 < NCH)
                def _():
                    pltpu.async_copy(lbl_hbm.at[pl.ds((ci + 32) * K, K)],
                                     lbp, sem_l[p])

        for p in range(2):
            @pl.when(st[p] == 1)
            def _(p=p):
                pltpu.make_async_copy(x_hbm.at[pl.ds(0, K)], xb[p],
                                      sem_s[p]).wait()
                pltpu.make_async_copy(x_hbm.at[pl.ds(0, K)], sq[p],
                                      sem_s[p]).wait()

        plsc.subcore_barrier()
        pltpu.sync_copy(hist, acc_c.at[iota], add=True)
        plsc.subcore_barrier()

        for h in range(RPT // K):
            rr = r0 + h * K
            pltpu.sync_copy(acc_s.at[pl.ds(rr, K)], xb0)
            pltpu.sync_copy(xb0, sum_hbm.at[core, pl.ds(rr, K)])
            pltpu.sync_copy(acc_q.at[pl.ds(rr, K)], xb1)
            pltpu.sync_copy(xb1, sq_hbm.at[core, pl.ds(rr, K)])

        @pl.when(sub == 0)
        def _():
            pltpu.sync_copy(acc_c.at[pl.ds(0, HROW)], hist)
            pltpu.sync_copy(hist, cnt_hbm.at[core])

    return k(x, labels)


_RB = 2560


def _mm_body(x_ref, w_ref, b_ref, o_ref):
    o_ref[...] = jnp.dot(x_ref[...], w_ref[...],
                         preferred_element_type=jnp.float32) + b_ref[...]


def _matmul(x, w, b2):
    return pl.pallas_call(
        _mm_body,
        grid=(N // _RB,),
        in_specs=[pl.BlockSpec((_RB, D), lambda i: (i, 0)),
                  pl.BlockSpec((D, D), lambda i: (0, 0)),
                  pl.BlockSpec((1, D), lambda i: (0, 0))],
        out_specs=pl.BlockSpec((_RB, D), lambda i: (i, 0)),
        out_shape=jax.ShapeDtypeStruct((N, D), jnp.float32),
    )(x, w, b2)


_FB = 2000


def _fin_body(s_ref, q_ref, c1_ref, rm_ref, rv_ref, cc_ref,
              om_ref, ov_ref, oc_ref):
    cnt = c1_ref[...]
    safe_new = jnp.maximum(cnt, 1.0)
    mean_new = s_ref[...] / safe_new
    var_new = q_ref[...] / safe_new - mean_new * mean_new
    pos = cnt > 0
    mean_new = jnp.where(pos, mean_new, 0.0)
    var_new = jnp.where(pos, var_new, 0.0)
    n_a = cc_ref[...]
    rm = rm_ref[...]
    nn = n_a + cnt
    safe = jnp.maximum(nn, 1.0)
    mean = (n_a * rm + cnt * mean_new) / safe
    var = (n_a * (rv_ref[...] + (rm - mean) ** 2)
           + cnt * (var_new + (mean_new - mean) ** 2)) / safe
    ok = nn > 0
    om_ref[...] = jnp.where(ok, mean, 0.0)
    ov_ref[...] = jnp.where(ok, var, 0.0)
    oc_ref[...] = nn


def _finalize(sums, sqs, c1, running_mean, running_var, class_count):
    return pl.pallas_call(
        _fin_body,
        grid=(C // _FB,),
        in_specs=[pl.BlockSpec((_FB, D), lambda i: (i, 0)),
                  pl.BlockSpec((_FB, D), lambda i: (i, 0)),
                  pl.BlockSpec((_FB, 1), lambda i: (i, 0)),
                  pl.BlockSpec((_FB, D), lambda i: (i, 0)),
                  pl.BlockSpec((_FB, D), lambda i: (i, 0)),
                  pl.BlockSpec((_FB, 1), lambda i: (i, 0))],
        out_specs=[pl.BlockSpec((_FB, D), lambda i: (i, 0)),
                   pl.BlockSpec((_FB, D), lambda i: (i, 0)),
                   pl.BlockSpec((_FB, 1), lambda i: (i, 0))],
        out_shape=[jax.ShapeDtypeStruct((C, D), jnp.float32),
                   jax.ShapeDtypeStruct((C, D), jnp.float32),
                   jax.ShapeDtypeStruct((C, 1), jnp.float32)],
    )(sums, sqs, c1, running_mean, running_var, class_count)


def kernel(x, labels, W, b, running_mean, running_var, class_count):
    labels = labels.astype(jnp.int32)
    out = _matmul(x, W, b.reshape(1, D))
    sums2, sqs2, cnt2 = _sc_stats(x, labels)
    sums = jnp.concatenate([sums2[0, :HALF], sums2[1, 1:HALF + 1]], axis=0)
    sqs = jnp.concatenate([sqs2[0, :HALF], sqs2[1, 1:HALF + 1]], axis=0)
    cflat0 = cnt2[0].reshape(HROW * 128)[:HALF]
    cflat1 = cnt2[1].reshape(HROW * 128)[1:HALF + 1]
    c1 = jnp.concatenate([cflat0, cflat1], axis=0)[:, None]
    rm, rv, cc = _finalize(sums, sqs, c1, running_mean, running_var,
                           class_count)
    return out, rm, rv, cc
